# Initial kernel scaffold; baseline (speedup 1.0000x reference)
import jax
import jax.numpy as jnp
from jax.experimental import pallas as pl


def kernel(x, W_router, W_gate, W_up, W_down):
    raise NotImplementedError("write your pallas kernel here")


# speedup vs baseline: 2.8202x; 2.8202x over previous
import jax
import jax.numpy as jnp
from jax.experimental import pallas as pl
from jax.experimental.pallas import tpu as pltpu

EMBED = 768
NEXP = 8
NTOK = 2048
DFF = 3072
BD = 512
NJ = DFF // BD


def _router_body(x_ref, wr_ref, w8_ref, lb_ref):
    x = x_ref[...]
    logits = jnp.dot(x, wr_ref[...], preferred_element_type=jnp.float32)
    m = jnp.max(logits, axis=1, keepdims=True)
    el = jnp.exp(logits - m)
    p = el / jnp.sum(el, axis=1, keepdims=True)
    idx8 = jax.lax.broadcasted_iota(jnp.int32, (NTOK, NEXP), 1)
    m1 = jnp.max(p, axis=1, keepdims=True)
    i1 = jnp.min(jnp.where(p == m1, idx8, NEXP), axis=1, keepdims=True)
    sel1 = idx8 == i1
    p2 = jnp.where(sel1, -1.0, p)
    m2 = jnp.max(p2, axis=1, keepdims=True)
    i2 = jnp.min(jnp.where(p2 == m2, idx8, NEXP), axis=1, keepdims=True)
    sel2 = idx8 == i2
    s = m1 + m2 + 1e-10
    w8_ref[...] = jnp.where(sel1, m1 / s, jnp.where(sel2, m2 / s, 0.0))
    ep = jnp.mean(p, axis=0, keepdims=True)
    lb = NEXP * jnp.sum(ep * jnp.log(ep * NEXP + 1e-10))
    lb_ref[...] = jnp.reshape(lb, (1, 1))


def _moe_body(x_ref, wg_ref, wu_ref, wd_ref, w8_ref, out_ref, acc_ref):
    j = pl.program_id(0)
    e = pl.program_id(1)

    @pl.when(jnp.logical_and(j == 0, e == 0))
    def _():
        out_ref[...] = jnp.zeros_like(out_ref)

    x = x_ref[...]
    g = jnp.dot(x, wg_ref[...].astype(jnp.bfloat16),
                preferred_element_type=jnp.float32)
    u = jnp.dot(x, wu_ref[...].astype(jnp.bfloat16),
                preferred_element_type=jnp.float32)
    h = g * jax.lax.logistic(g) * u
    ohe = (jax.lax.broadcasted_iota(jnp.int32, (1, NEXP), 1) == e
           ).astype(jnp.float32)
    wcol = jnp.sum(w8_ref[...] * ohe, axis=1, keepdims=True)
    h = h * wcol

    @pl.when(e == 0)
    def _():
        acc_ref[...] = h

    @pl.when(e != 0)
    def _():
        acc_ref[...] += h

    @pl.when(e == NEXP - 1)
    def _():
        out_ref[...] += jnp.dot(acc_ref[...].astype(jnp.bfloat16),
                                wd_ref[...].astype(jnp.bfloat16),
                                preferred_element_type=jnp.float32)


def _router_call(x_flat, W_router):
    return pl.pallas_call(
        _router_body,
        out_shape=(
            jax.ShapeDtypeStruct((NTOK, NEXP), jnp.float32),
            jax.ShapeDtypeStruct((1, 1), jnp.float32),
        ),
    )(x_flat, W_router)


def _moe_call(x_bf, W_gate, W_up, W_down, w8):
    return pl.pallas_call(
        _moe_body,
        grid=(NJ, NEXP),
        in_specs=[
            pl.BlockSpec((NTOK, EMBED), lambda j, e: (0, 0)),
            pl.BlockSpec((EMBED, BD), lambda j, e: (0, e * NJ + j)),
            pl.BlockSpec((EMBED, BD), lambda j, e: (0, e * NJ + j)),
            pl.BlockSpec((BD, EMBED), lambda j, e: (j, 0)),
            pl.BlockSpec((NTOK, NEXP), lambda j, e: (0, 0)),
        ],
        out_specs=pl.BlockSpec((NTOK, EMBED), lambda j, e: (0, 0)),
        out_shape=jax.ShapeDtypeStruct((NTOK, EMBED), jnp.float32),
        scratch_shapes=[pltpu.VMEM((NTOK, BD), jnp.float32)],
    )(x_bf, W_gate, W_up, W_down, w8)


def kernel(x, W_router, W_gate, W_up, W_down):
    x_flat = x.reshape(NTOK, EMBED)
    w8, lb = _router_call(x_flat, W_router)
    x_bf = x_flat.astype(jnp.bfloat16)
    out = _moe_call(x_bf, W_gate, W_up, W_down, w8)
    return out.reshape(x.shape), lb[0, 0]
